# baseline (speedup 1.0000x reference)
import jax
import jax.numpy as jnp
from jax.experimental import pallas as pl
from jax.experimental.pallas import tpu as pltpu

EPS_DICE = 0.0001
NC = 8
BB = 1


def _dice_body(*refs):
    xs = refs[:NC]
    tgt_ref = refs[NC]
    loss_ref = refs[NC + 1]

    for b2 in range(BB):
        t = tgt_ref[b2]
        h, w = t.shape
        npix = h * w * 1.0

        n_int = [jnp.zeros((w,), jnp.int32) for _ in range(NC)]
        qrows = 128
        for q in range(h // qrows):
            tq = t[q * qrows:(q + 1) * qrows, :]
            sh = jnp.left_shift(1, (tq & 3) << 3)
            is_lo = tq < 4
            lo = jnp.sum(jnp.where(is_lo, sh, 0), axis=0)
            hi = jnp.sum(jnp.where(is_lo, 0, sh), axis=0)
            for f in range(4):
                n_int[f] = n_int[f] + ((lo >> (8 * f)) & 255)
                n_int[4 + f] = n_int[4 + f] + ((hi >> (8 * f)) & 255)

        a_list = []
        bn_list = []
        for c in range(NC):
            x = xs[c][b2, 0]
            th = jnp.tanh(x * 0.5)
            eq = t == c
            a_list.append(jnp.sum(th, axis=0))
            bn_list.append(jnp.sum(jnp.where(eq, th, 0.0), axis=0))

        a = jnp.stack([jnp.sum(v) for v in a_list])
        bb = jnp.stack([jnp.sum(v) for v in bn_list])
        n = jnp.stack([jnp.sum(v).astype(jnp.float32) for v in n_int])
        s = 0.5 * (a + npix)
        tt = 0.5 * (bb + n)
        per_class = 1.0 - 2.0 * tt / (s + n + EPS_DICE)
        loss_ref[b2, 0, :] = jnp.full((loss_ref.shape[-1],), jnp.sum(per_class) / NC)


def kernel(output, target):
    b, nc, h, w = output.shape
    tgt = target.astype(jnp.int32)

    def chan_spec(c):
        return pl.BlockSpec((BB, 1, h, w), lambda bi, cc=c: (bi, cc, 0, 0))

    padded = pl.pallas_call(
        _dice_body,
        grid=(b // BB,),
        in_specs=[chan_spec(c) for c in range(nc)]
        + [pl.BlockSpec((BB, h, w), lambda bi: (bi, 0, 0))],
        out_specs=pl.BlockSpec((BB, 1, 128), lambda bi: (bi, 0, 0)),
        out_shape=jax.ShapeDtypeStruct((b, 1, 128), jnp.float32),
    )(*([output] * nc + [tgt]))
    return padded[:, 0, 0]


# speedup vs baseline: 1.1133x; 1.1133x over previous
import jax
import jax.numpy as jnp
from jax.experimental import pallas as pl
from jax.experimental.pallas import tpu as pltpu

EPS_DICE = 0.0001
NC = 8
BB = 2


def _dice_body(*refs):
    xs = refs[:NC]
    tgt_ref = refs[NC]
    loss_ref = refs[NC + 1]

    for b2 in range(BB):
        t = tgt_ref[b2]
        h, w = t.shape
        npix = h * w * 1.0

        n_int = [jnp.zeros((w,), jnp.int32) for _ in range(NC)]
        qrows = 128
        for q in range(h // qrows):
            tq = t[q * qrows:(q + 1) * qrows, :]
            sh = jnp.left_shift(1, (tq & 3) << 3)
            is_lo = tq < 4
            lo = jnp.sum(jnp.where(is_lo, sh, 0), axis=0)
            hi = jnp.sum(jnp.where(is_lo, 0, sh), axis=0)
            for f in range(4):
                n_int[f] = n_int[f] + ((lo >> (8 * f)) & 255)
                n_int[4 + f] = n_int[4 + f] + ((hi >> (8 * f)) & 255)

        ones8 = jnp.ones((8, h), jnp.float32)
        a_list = []
        bn_list = []
        for c in range(NC):
            x = xs[c][b2, 0]
            th = jnp.tanh(x * 0.5)
            eq = t == c
            masked = jnp.where(eq, th, 0.0)
            a_list.append(jnp.dot(ones8, th)[0])
            bn_list.append(jnp.dot(ones8, masked)[0])

        a = jnp.stack([jnp.sum(v) for v in a_list])
        bb = jnp.stack([jnp.sum(v) for v in bn_list])
        n = jnp.stack([jnp.sum(v).astype(jnp.float32) for v in n_int])
        s = 0.5 * (a + npix)
        tt = 0.5 * (bb + n)
        per_class = 1.0 - 2.0 * tt / (s + n + EPS_DICE)
        loss_ref[b2, 0, :] = jnp.full((loss_ref.shape[-1],), jnp.sum(per_class) / NC)


def kernel(output, target):
    b, nc, h, w = output.shape
    tgt = target.astype(jnp.int32)

    def chan_spec(c):
        return pl.BlockSpec((BB, 1, h, w), lambda bi, cc=c: (bi, cc, 0, 0))

    padded = pl.pallas_call(
        _dice_body,
        grid=(b // BB,),
        in_specs=[chan_spec(c) for c in range(nc)]
        + [pl.BlockSpec((BB, h, w), lambda bi: (bi, 0, 0))],
        out_specs=pl.BlockSpec((BB, 1, 128), lambda bi: (bi, 0, 0)),
        out_shape=jax.ShapeDtypeStruct((b, 1, 128), jnp.float32),
    )(*([output] * nc + [tgt]))
    return padded[:, 0, 0]


# speedup vs baseline: 1.1146x; 1.0012x over previous
import jax
import jax.numpy as jnp
from jax.experimental import pallas as pl
from jax.experimental.pallas import tpu as pltpu

EPS_DICE = 0.0001
NC = 8
BB = 2


def _dice_body(*refs):
    xs = refs[:NC]
    tgt_ref = refs[NC]
    loss_ref = refs[NC + 1]

    for b2 in range(BB):
        t = tgt_ref[b2]
        h, w = t.shape
        npix = h * w * 1.0

        n_int = [jnp.zeros((w,), jnp.int32) for _ in range(NC)]
        qrows = 128
        for q in range(h // qrows):
            tq = t[q * qrows:(q + 1) * qrows, :]
            sh = jnp.left_shift(1, (tq & 3) << 3)
            is_lo = tq < 4
            lo = jnp.sum(jnp.where(is_lo, sh, 0), axis=0)
            hi = jnp.sum(jnp.where(is_lo, 0, sh), axis=0)
            for f in range(4):
                n_int[f] = n_int[f] + ((lo >> (8 * f)) & 255)
                n_int[4 + f] = n_int[4 + f] + ((hi >> (8 * f)) & 255)

        ones8 = jnp.ones((8, h), jnp.float32)
        a_list = []
        bn_list = []
        for c in range(NC):
            x = xs[c][b2, 0]
            th = jnp.tanh(x * 0.5)
            eq = t == c
            masked = jnp.where(eq, th, 0.0)
            a_list.append(jnp.dot(ones8, th)[0])
            bn_list.append(jnp.dot(ones8, masked)[0])

        a = jnp.stack([jnp.sum(v) for v in a_list])
        bb = jnp.stack([jnp.sum(v) for v in bn_list])
        n = jnp.stack([jnp.sum(v).astype(jnp.float32) for v in n_int])
        s = 0.5 * (a + npix)
        tt = 0.5 * (bb + n)
        per_class = 1.0 - 2.0 * tt / (s + n + EPS_DICE)
        loss_ref[b2, 0, :] = jnp.full((loss_ref.shape[-1],), jnp.sum(per_class) / NC)


def kernel(output, target):
    b, nc, h, w = output.shape
    tgt = target.astype(jnp.int32)

    def chan_spec(c):
        return pl.BlockSpec((BB, 1, h, w), lambda bi, cc=c: (bi, cc, 0, 0))

    padded = pl.pallas_call(
        _dice_body,
        grid=(b // BB,),
        in_specs=[chan_spec(c) for c in range(nc)]
        + [pl.BlockSpec((BB, h, w), lambda bi: (bi, 0, 0))],
        out_specs=pl.BlockSpec((BB, 1, 128), lambda bi: (bi, 0, 0)),
        out_shape=jax.ShapeDtypeStruct((b, 1, 128), jnp.float32),
        compiler_params=pltpu.CompilerParams(
            dimension_semantics=("parallel",),
        ),
    )(*([output] * nc + [tgt]))
    return padded[:, 0, 0]


# speedup vs baseline: 1.1188x; 1.0038x over previous
import jax
import jax.numpy as jnp
from jax.experimental import pallas as pl
from jax.experimental.pallas import tpu as pltpu

EPS_DICE = 0.0001
NC = 8
BB = 2


def _dice_body(*refs):
    xs = refs[:2 * NC]
    tgt_ref = refs[2 * NC]
    loss_ref = refs[2 * NC + 1]

    for b2 in range(BB):
        t = tgt_ref[b2]
        h, w = t.shape
        npix = h * w * 1.0

        n_int = [jnp.zeros((w,), jnp.int32) for _ in range(NC)]
        qrows = 128
        for q in range(h // qrows):
            tq = t[q * qrows:(q + 1) * qrows, :]
            sh = jnp.left_shift(1, (tq & 3) << 3)
            is_lo = tq < 4
            lo = jnp.sum(jnp.where(is_lo, sh, 0), axis=0)
            hi = jnp.sum(jnp.where(is_lo, 0, sh), axis=0)
            for f in range(4):
                n_int[f] = n_int[f] + ((lo >> (8 * f)) & 255)
                n_int[4 + f] = n_int[4 + f] + ((hi >> (8 * f)) & 255)

        ones8 = jnp.ones((8, h // 2), jnp.float32)
        a_list = []
        bn_list = []
        for c in range(NC):
            av = None
            bv = None
            for hh in range(2):
                x = xs[2 * c + hh][b2, 0]
                th = jnp.tanh(x * 0.5)
                eq = t[hh * (h // 2):(hh + 1) * (h // 2), :] == c
                masked = jnp.where(eq, th, 0.0)
                ac = jnp.dot(ones8, th)[0]
                bc = jnp.dot(ones8, masked)[0]
                av = ac if av is None else av + ac
                bv = bc if bv is None else bv + bc
            a_list.append(av)
            bn_list.append(bv)

        a = jnp.stack([jnp.sum(v) for v in a_list])
        bb = jnp.stack([jnp.sum(v) for v in bn_list])
        n = jnp.stack([jnp.sum(v).astype(jnp.float32) for v in n_int])
        s = 0.5 * (a + npix)
        tt = 0.5 * (bb + n)
        per_class = 1.0 - 2.0 * tt / (s + n + EPS_DICE)
        loss_ref[b2, 0, :] = jnp.full((loss_ref.shape[-1],), jnp.sum(per_class) / NC)


def kernel(output, target):
    b, nc, h, w = output.shape
    tgt = target.astype(jnp.int32)

    def chan_spec(c, hh):
        return pl.BlockSpec(
            (BB, 1, h // 2, w), lambda bi, cc=c, h2=hh: (bi, cc, h2, 0)
        )

    padded = pl.pallas_call(
        _dice_body,
        grid=(b // BB,),
        in_specs=[chan_spec(c, hh) for c in range(nc) for hh in range(2)]
        + [pl.BlockSpec((BB, h, w), lambda bi: (bi, 0, 0))],
        out_specs=pl.BlockSpec((BB, 1, 128), lambda bi: (bi, 0, 0)),
        out_shape=jax.ShapeDtypeStruct((b, 1, 128), jnp.float32),
    )(*([output] * (2 * nc) + [tgt]))
    return padded[:, 0, 0]
